# Initial kernel scaffold; baseline (speedup 1.0000x reference)
import jax
import jax.numpy as jnp
from jax.experimental import pallas as pl


def kernel(cate_x, emb_table, W, b, gamma, beta):
    raise NotImplementedError("write your pallas kernel here")


# speedup vs baseline: 14.6499x; 14.6499x over previous
import functools

import jax
import jax.numpy as jnp
from jax import lax
from jax.experimental import pallas as pl
from jax.experimental.pallas import tpu as pltpu
from jax.experimental.pallas import tpu_sc as plsc

EMB_DIM = 32
CATE_NUM = 26
PROJ_DIM = 128
MAX_SEQ_LEN = 50
BATCH = 1024

N_ROWS = BATCH * MAX_SEQ_LEN * CATE_NUM
IN_DIM = EMB_DIM * CATE_NUM
BL = BATCH * MAX_SEQ_LEN

NUM_CORES = 2
NUM_SUBCORES = 16
NUM_TILES = NUM_CORES * NUM_SUBCORES
CHUNK = 128
GROUP = 5
ROWS_PER_TILE = N_ROWS // NUM_TILES
CHUNKS_PER_TILE = ROWS_PER_TILE // CHUNK
GROUPS_PER_TILE = CHUNKS_PER_TILE // GROUP
GROUP_ROWS = GROUP * CHUNK


def _sc_gather(idx3d, table):
    mesh = plsc.VectorSubcoreMesh(core_axis_name="c", subcore_axis_name="s")

    @functools.partial(
        pl.kernel,
        mesh=mesh,
        compiler_params=pltpu.CompilerParams(use_tc_tiling_on_sc=False),
        out_type=jax.ShapeDtypeStruct((N_ROWS, EMB_DIM), jnp.float32),
        scratch_types=[
            pltpu.VMEM((CHUNKS_PER_TILE, CHUNK), jnp.int32),
            pltpu.VMEM((GROUP_ROWS, EMB_DIM), jnp.float32),
            pltpu.SemaphoreType.DMA,
        ],
    )
    def k(idx_hbm, table_hbm, out_hbm, idx_v, rows_v, sem):
        wid = lax.axis_index("s") * NUM_CORES + lax.axis_index("c")
        pltpu.sync_copy(idx_hbm.at[wid], idx_v)
        out_base = wid * ROWS_PER_TILE

        def body(g, carry):
            cps = []
            for j in range(GROUP):
                cps.append(pltpu.async_copy(
                    table_hbm.at[idx_v.at[g * GROUP + j]],
                    rows_v.at[pl.ds(j * CHUNK, CHUNK)],
                    sem))
            for cp in cps:
                cp.wait()
            pltpu.sync_copy(
                rows_v,
                out_hbm.at[pl.ds(out_base + g * GROUP_ROWS, GROUP_ROWS)])
            return carry

        lax.fori_loop(0, GROUPS_PER_TILE, body, 0)

    return k(idx3d, table)


def _tc_proj_body(x_ref, w_ref, b_ref, g_ref, be_ref, o_ref):
    h = jnp.dot(x_ref[...], w_ref[...], preferred_element_type=jnp.float32)
    h = h + b_ref[...]
    mu = jnp.mean(h, axis=1, keepdims=True)
    d = h - mu
    var = jnp.mean(d * d, axis=1, keepdims=True)
    o_ref[...] = d * lax.rsqrt(var + 1e-5) * g_ref[...] + be_ref[...]


def _tc_proj(x, W, b, gamma, beta):
    TM = 512
    grid = (BL // TM,)
    return pl.pallas_call(
        _tc_proj_body,
        grid=grid,
        in_specs=[
            pl.BlockSpec((TM, IN_DIM), lambda i: (i, 0)),
            pl.BlockSpec((IN_DIM, PROJ_DIM), lambda i: (0, 0)),
            pl.BlockSpec((1, PROJ_DIM), lambda i: (0, 0)),
            pl.BlockSpec((1, PROJ_DIM), lambda i: (0, 0)),
            pl.BlockSpec((1, PROJ_DIM), lambda i: (0, 0)),
        ],
        out_specs=pl.BlockSpec((TM, PROJ_DIM), lambda i: (i, 0)),
        out_shape=jax.ShapeDtypeStruct((BL, PROJ_DIM), jnp.float32),
    )(x, W, b.reshape(1, PROJ_DIM), gamma.reshape(1, PROJ_DIM),
      beta.reshape(1, PROJ_DIM))


def kernel(cate_x, emb_table, W, b, gamma, beta):
    idx3d = cate_x.reshape(NUM_TILES, CHUNKS_PER_TILE, CHUNK)
    gathered = _sc_gather(idx3d, emb_table)
    x = gathered.reshape(BL, IN_DIM)
    out = _tc_proj(x, W, b, gamma, beta)
    return out.reshape(BATCH, MAX_SEQ_LEN, PROJ_DIM)


# speedup vs baseline: 20.7785x; 1.4183x over previous
import functools

import jax
import jax.numpy as jnp
from jax import lax
from jax.experimental import pallas as pl
from jax.experimental.pallas import tpu as pltpu
from jax.experimental.pallas import tpu_sc as plsc

EMB_DIM = 32
CATE_NUM = 26
PROJ_DIM = 128
MAX_SEQ_LEN = 50
BATCH = 1024

N_ROWS = BATCH * MAX_SEQ_LEN * CATE_NUM
IN_DIM = EMB_DIM * CATE_NUM
BL = BATCH * MAX_SEQ_LEN
NW128 = N_ROWS * EMB_DIM // 128

NUM_CORES = 2
NUM_SUBCORES = 16
NUM_TILES = NUM_CORES * NUM_SUBCORES
CHUNK = 128
GROUP = 5
ROWS_PER_TILE = N_ROWS // NUM_TILES
CHUNKS_PER_TILE = ROWS_PER_TILE // CHUNK
GROUPS_PER_TILE = CHUNKS_PER_TILE // GROUP
GROUP_ROWS = GROUP * CHUNK


def _sc_gather(idx3d, table):
    mesh = plsc.VectorSubcoreMesh(core_axis_name="c", subcore_axis_name="s")

    @functools.partial(
        pl.kernel,
        mesh=mesh,
        compiler_params=pltpu.CompilerParams(use_tc_tiling_on_sc=False),
        out_type=jax.ShapeDtypeStruct((N_ROWS, EMB_DIM), jnp.float32),
        scratch_types=[
            pltpu.VMEM((CHUNKS_PER_TILE, CHUNK), jnp.int32),
            pltpu.VMEM((GROUP_ROWS, EMB_DIM), jnp.float32),
            pltpu.SemaphoreType.DMA,
        ],
    )
    def k(idx_hbm, table_hbm, out_hbm, idx_v, rows_v, sem):
        wid = lax.axis_index("s") * NUM_CORES + lax.axis_index("c")
        pltpu.sync_copy(idx_hbm.at[wid], idx_v)
        out_base = wid * ROWS_PER_TILE

        def body(g, carry):
            cps = []
            for j in range(GROUP):
                cps.append(pltpu.async_copy(
                    table_hbm.at[idx_v.at[g * GROUP + j]],
                    rows_v.at[pl.ds(j * CHUNK, CHUNK)],
                    sem))
            for cp in cps:
                cp.wait()
            pltpu.sync_copy(
                rows_v,
                out_hbm.at[pl.ds(out_base + g * GROUP_ROWS, GROUP_ROWS)])
            return carry

        lax.fori_loop(0, GROUPS_PER_TILE, body, 0)

    return k(idx3d, table)


BB = 16
PAIRS = BB * MAX_SEQ_LEN // 2
XROWS = PAIRS * 13


def _tc_proj_body(x_ref, w_ref, b_ref, g_ref, be_ref, o_ref):
    x2 = x_ref[...].reshape(PAIRS, 2 * IN_DIM)
    w = w_ref[...]
    h_even = jnp.dot(x2[:, :IN_DIM], w, preferred_element_type=jnp.float32)
    h_odd = jnp.dot(x2[:, IN_DIM:], w, preferred_element_type=jnp.float32)
    h = jnp.stack([h_even, h_odd], axis=1).reshape(2 * PAIRS, PROJ_DIM)
    h = h + b_ref[...]
    mu = jnp.mean(h, axis=1, keepdims=True)
    d = h - mu
    var = jnp.mean(d * d, axis=1, keepdims=True)
    out = d * lax.rsqrt(var + 1e-5) * g_ref[...] + be_ref[...]
    o_ref[...] = out.reshape(BB, MAX_SEQ_LEN, PROJ_DIM)


def _tc_proj(x128, W, b, gamma, beta):
    grid = (BATCH // BB,)
    return pl.pallas_call(
        _tc_proj_body,
        grid=grid,
        in_specs=[
            pl.BlockSpec((XROWS, 128), lambda i: (i, 0)),
            pl.BlockSpec((IN_DIM, PROJ_DIM), lambda i: (0, 0)),
            pl.BlockSpec((1, PROJ_DIM), lambda i: (0, 0)),
            pl.BlockSpec((1, PROJ_DIM), lambda i: (0, 0)),
            pl.BlockSpec((1, PROJ_DIM), lambda i: (0, 0)),
        ],
        out_specs=pl.BlockSpec((BB, MAX_SEQ_LEN, PROJ_DIM), lambda i: (i, 0, 0)),
        out_shape=jax.ShapeDtypeStruct((BATCH, MAX_SEQ_LEN, PROJ_DIM),
                                       jnp.float32),
    )(x128, W, b.reshape(1, PROJ_DIM), gamma.reshape(1, PROJ_DIM),
      beta.reshape(1, PROJ_DIM))


def kernel(cate_x, emb_table, W, b, gamma, beta):
    idx3d = cate_x.reshape(NUM_TILES, CHUNKS_PER_TILE, CHUNK)
    gathered = _sc_gather(idx3d, emb_table)
    x128 = gathered.reshape(NW128, 128)
    return _tc_proj(x128, W, b, gamma, beta)
